# Initial kernel scaffold; baseline (speedup 1.0000x reference)
import jax
import jax.numpy as jnp
from jax.experimental import pallas as pl


def kernel(vals, W, coords):
    raise NotImplementedError("write your pallas kernel here")


# speedup vs baseline: 3.5017x; 3.5017x over previous
import functools

import jax
import jax.numpy as jnp
from jax import lax
from jax.experimental import pallas as pl
from jax.experimental.pallas import tpu as pltpu
from jax.experimental.pallas import tpu_sc as plsc

NVOX = 96
VOX3 = NVOX * NVOX * NVOX
N = 200000
CDIM = 32
ADIM = 23

L = 16
NC = 2
NSC = 16
NW = NC * NSC
RPW = 6272
NP = NW * RPW
VPW = VOX3 // NW
CHUNK = 6272
NCHUNK = NP // CHUNK
GCH = 896
NGCH = RPW // GCH

_MESH = plsc.VectorSubcoreMesh(core_axis_name="c", subcore_axis_name="s")
_SC_PARAMS = pltpu.CompilerParams(needs_layout_passes=False)
_SC_PARAMS_LINEAR = pltpu.CompilerParams(
    needs_layout_passes=False, use_tc_tiling_on_sc=False
)


def _wid():
  return lax.axis_index("s") * NC + lax.axis_index("c")


@functools.partial(
    pl.kernel,
    out_type=jax.ShapeDtypeStruct((NP,), jnp.int32),
    mesh=_MESH,
    compiler_params=_SC_PARAMS,
    scratch_types=[
        pltpu.VMEM((RPW * 3,), jnp.int32),
        pltpu.VMEM((RPW,), jnp.int32),
    ],
)
def _flat_kernel(coords_hbm, flat_hbm, cbuf, fbuf):
  base = _wid() * RPW
  pltpu.sync_copy(coords_hbm.at[pl.ds(base * 3, RPW * 3)], cbuf)
  lanes = lax.iota(jnp.int32, L)

  @plsc.parallel_loop(0, RPW // L, unroll=8)
  def _(k):
    r3 = (k * L + lanes) * 3
    x = plsc.load_gather(cbuf, [r3])
    y = plsc.load_gather(cbuf, [r3 + 1])
    z = plsc.load_gather(cbuf, [r3 + 2])
    fbuf[pl.ds(k * L, L)] = x * (NVOX * NVOX) + y * NVOX + z

  pltpu.sync_copy(fbuf, flat_hbm.at[pl.ds(base, RPW)])


@functools.partial(
    pl.kernel,
    out_type=jax.ShapeDtypeStruct((VOX3,), jnp.int32),
    mesh=_MESH,
    compiler_params=_SC_PARAMS,
    scratch_types=[
        pltpu.VMEM((VPW,), jnp.int32),
        pltpu.VMEM((CHUNK,), jnp.int32),
    ],
)
def _winner_kernel(flat_hbm, win_hbm, slc, fstage):
  wid = _wid()
  lo = wid * VPW
  lanes = lax.iota(jnp.int32, L)
  zeros = jnp.zeros((L,), jnp.int32)

  @plsc.parallel_loop(0, VPW // L, unroll=8)
  def _(k):
    slc[pl.ds(k * L, L)] = zeros

  def chunk_body(c, carry):
    pltpu.sync_copy(flat_hbm.at[pl.ds(c * CHUNK, CHUNK)], fstage)

    def vec_body(k, carry2):
      f = fstage[pl.ds(k * L, L)]
      rid = c * CHUNK + k * L + lanes
      m = (f >= lo) & (f < lo + VPW) & (rid < N)
      _, lastm = plsc.scan_count(f, m)
      loc = jnp.where(m, f - lo, 0)
      plsc.store_scatter(slc, [loc], rid, mask=lastm & m)
      return carry2

    return lax.fori_loop(0, CHUNK // L, vec_body, carry, unroll=4)

  lax.fori_loop(0, NCHUNK, chunk_body, 0)
  pltpu.sync_copy(slc, win_hbm.at[pl.ds(lo, VPW)])


@functools.partial(
    pl.kernel,
    out_type=jax.ShapeDtypeStruct((NP, CDIM), jnp.float32),
    mesh=_MESH,
    compiler_params=_SC_PARAMS_LINEAR,
    scratch_types=[
        pltpu.VMEM((RPW,), jnp.int32),
        pltpu.VMEM((RPW,), jnp.int32),
        pltpu.VMEM((GCH, CDIM), jnp.float32),
        pltpu.SemaphoreType.DMA,
    ],
)
def _gather_kernel(flat_hbm, win_hbm, vals_hbm, out_hbm, fidx, wrow, rows, sem):
  base = _wid() * RPW
  pltpu.sync_copy(flat_hbm.at[pl.ds(base, RPW)], fidx)
  pltpu.async_copy(win_hbm.at[fidx], wrow, sem).wait()
  for cc in range(NGCH):
    pltpu.async_copy(
        vals_hbm.at[wrow.at[pl.ds(cc * GCH, GCH)]], rows, sem
    ).wait()
    pltpu.sync_copy(rows, out_hbm.at[pl.ds(base + cc * GCH, GCH)])


_TC_BLK = 2000


@functools.partial(
    pl.pallas_call,
    out_shape=jax.ShapeDtypeStruct((N, ADIM), jnp.float32),
    grid=(N // _TC_BLK,),
    in_specs=[
        pl.BlockSpec((_TC_BLK, CDIM), lambda i: (i, 0)),
        pl.BlockSpec((CDIM, ADIM), lambda i: (0, 0)),
    ],
    out_specs=pl.BlockSpec((_TC_BLK, ADIM), lambda i: (i, 0)),
)
def _matmul_relu(x_ref, w_ref, o_ref):
  o_ref[...] = jnp.maximum(
      jnp.dot(x_ref[...], w_ref[...], preferred_element_type=jnp.float32), 0.0
  )


def kernel(vals, W, coords):
  coords_p = jnp.pad(coords, ((0, NP - N), (0, 0)))
  flat = _flat_kernel(coords_p.reshape(-1))
  win = _winner_kernel(flat)
  gathered = _gather_kernel(flat, win, vals)
  return _matmul_relu(gathered, W)
